# baseline (speedup 1.0000x reference)
import jax
import jax.numpy as jnp
from jax import lax
from jax.experimental import pallas as pl
from jax.experimental.pallas import tpu as pltpu
from jax.experimental.pallas import tpu_sc as plsc

BATCH = 4096
SEQ = 200
EMBED = 64
VOCAB = 1000000
LANES = 16

NUM_CORES = 2
NUM_SUBCORES = 16
NW = NUM_CORES * NUM_SUBCORES
B_PER_W = BATCH // NW
CHUNK_A = 104
CHUNK_B = SEQ - CHUNK_A
IDX_PER_W = B_PER_W * SEQ
NVREG = EMBED // LANES
INV_SEQ = 1.0 / SEQ

PACK_C = 512
PACK_H = PACK_C // 2
NBLK = -(-VOCAB // PACK_C)
VOCAB_PAD = NBLK * PACK_C


def _pack_body(t_ref, out_ref):
    ta = jnp.transpose(t_ref[:, :PACK_H], (1, 0))
    tb = jnp.transpose(t_ref[:, PACK_H:], (1, 0))
    out_ref[...] = jnp.concatenate([ta, tb], axis=1)


def _sc_body(x_hbm, table_hbm, out_hbm, idx_v, buf0, buf1, out_v, sem0, sem1):
    wid = lax.axis_index("s") * NUM_CORES + lax.axis_index("c")
    obase = wid * B_PER_W

    pltpu.sync_copy(x_hbm.at[pl.ds(wid * IDX_PER_W, IDX_PER_W)], idx_v)

    def remap(r, _):
        v = idx_v[pl.ds(r * LANES, LANES)]
        m = ((v >> 9) << 9) + ((v & 255) << 1) + ((v >> 8) & 1)
        idx_v[pl.ds(r * LANES, LANES)] = m
        return 0

    lax.fori_loop(0, IDX_PER_W // LANES, remap, 0)

    def start(b, buf, sem):
        pltpu.async_copy(table_hbm.at[idx_v.at[pl.ds(SEQ * b, CHUNK_A)]],
                         buf.at[pl.ds(0, CHUNK_A)], sem)
        pltpu.async_copy(table_hbm.at[idx_v.at[pl.ds(SEQ * b + CHUNK_A, CHUNK_B)]],
                         buf.at[pl.ds(CHUNK_A, CHUNK_B)], sem)

    def wait(buf, sem):
        pltpu.make_async_copy(table_hbm.at[idx_v.at[pl.ds(0, CHUNK_A)]],
                              buf.at[pl.ds(0, CHUNK_A)], sem).wait()
        pltpu.make_async_copy(table_hbm.at[idx_v.at[pl.ds(0, CHUNK_B)]],
                              buf.at[pl.ds(CHUNK_A, CHUNK_B)], sem).wait()

    def accum_store(b, buf):
        def rbody(r4, acc):
            r = r4 * 4
            out = []
            for k in range(NVREG):
                s = buf[r, pl.ds(LANES * k, LANES)] + buf[r + 1, pl.ds(LANES * k, LANES)]
                t = buf[r + 2, pl.ds(LANES * k, LANES)] + buf[r + 3, pl.ds(LANES * k, LANES)]
                out.append(acc[k] + (s + t))
            return tuple(out)

        zero = jnp.zeros((LANES,), jnp.float32)
        acc = lax.fori_loop(0, SEQ // 4, rbody, (zero,) * NVREG)
        for k in range(NVREG):
            out_v[b, pl.ds(LANES * k, LANES)] = acc[k] * INV_SEQ

    start(0, buf0, sem0)

    def body(t, _):
        b0 = 2 * t
        start(b0 + 1, buf1, sem1)
        wait(buf0, sem0)
        accum_store(b0, buf0)

        @pl.when(b0 + 2 < B_PER_W)
        def _():
            start(b0 + 2, buf0, sem0)

        wait(buf1, sem1)
        accum_store(b0 + 1, buf1)
        return 0

    lax.fori_loop(0, B_PER_W // 2, body, 0)

    pltpu.sync_copy(out_v, out_hbm.at[pl.ds(obase, B_PER_W)])


@jax.jit
def _encode(x1, table_lin):
    mesh = plsc.VectorSubcoreMesh(core_axis_name="c", subcore_axis_name="s")
    return pl.kernel(
        _sc_body,
        out_type=jax.ShapeDtypeStruct((BATCH, EMBED), jnp.float32),
        mesh=mesh,
        compiler_params=pltpu.CompilerParams(use_tc_tiling_on_sc=False),
        scratch_types=[
            pltpu.VMEM((IDX_PER_W,), jnp.int32),
            pltpu.VMEM((SEQ, EMBED), jnp.float32),
            pltpu.VMEM((SEQ, EMBED), jnp.float32),
            pltpu.VMEM((B_PER_W, EMBED), jnp.float32),
            pltpu.SemaphoreType.DMA,
            pltpu.SemaphoreType.DMA,
        ],
    )(x1, table_lin)


@jax.jit
def _relayout(table):
    t_t = jnp.swapaxes(table, 0, 1)
    packed = pl.pallas_call(
        _pack_body,
        grid=(NBLK,),
        in_specs=[pl.BlockSpec((EMBED, PACK_C), lambda g: (0, g))],
        out_specs=pl.BlockSpec((PACK_H, 2 * EMBED), lambda g: (g, 0)),
        out_shape=jax.ShapeDtypeStruct((NBLK * PACK_H, 2 * EMBED), jnp.float32),
    )(t_t)
    return packed.reshape(VOCAB_PAD, EMBED)


def kernel(x, table):
    x1 = x.astype(jnp.int32).reshape(BATCH * SEQ)
    return _encode(x1, _relayout(table))


# speedup vs baseline: 2.8540x; 2.8540x over previous
import jax
import jax.numpy as jnp
from jax import lax
from jax.experimental import pallas as pl
from jax.experimental.pallas import tpu as pltpu
from jax.experimental.pallas import tpu_sc as plsc

BATCH = 4096
SEQ = 200
EMBED = 64
VOCAB = 1000000
LANES = 16

NUM_CORES = 2
NUM_SUBCORES = 16
NW = NUM_CORES * NUM_SUBCORES
B_PER_W = BATCH // NW
CHUNK_A = 104
CHUNK_B = SEQ - CHUNK_A
IDX_PER_W = B_PER_W * SEQ
NVREG = EMBED // LANES
INV_SEQ = 1.0 / SEQ

PACK_C = 4096
PACK_H = PACK_C // 2
NBLK = -(-VOCAB // PACK_C)
VOCAB_PAD = NBLK * PACK_C


def _pack_body(t_ref, out_ref):
    ta = jnp.transpose(t_ref[:, :PACK_H], (1, 0))
    tb = jnp.transpose(t_ref[:, PACK_H:], (1, 0))
    out_ref[...] = jnp.concatenate([ta, tb], axis=1)


def _sc_body(x_hbm, table_hbm, out_hbm, idx_v, buf0, buf1, out_v, sem0, sem1):
    wid = lax.axis_index("s") * NUM_CORES + lax.axis_index("c")
    obase = wid * B_PER_W

    pltpu.sync_copy(x_hbm.at[pl.ds(wid * IDX_PER_W, IDX_PER_W)], idx_v)

    def remap(r, _):
        v = idx_v[pl.ds(r * LANES, LANES)]
        m = ((v >> 12) << 12) + ((v & (PACK_H - 1)) << 1) + ((v >> 11) & 1)
        idx_v[pl.ds(r * LANES, LANES)] = m
        return 0

    lax.fori_loop(0, IDX_PER_W // LANES, remap, 0)

    def start(b, buf, sem):
        pltpu.async_copy(table_hbm.at[idx_v.at[pl.ds(SEQ * b, CHUNK_A)]],
                         buf.at[pl.ds(0, CHUNK_A)], sem)
        pltpu.async_copy(table_hbm.at[idx_v.at[pl.ds(SEQ * b + CHUNK_A, CHUNK_B)]],
                         buf.at[pl.ds(CHUNK_A, CHUNK_B)], sem)

    def wait(buf, sem):
        pltpu.make_async_copy(table_hbm.at[idx_v.at[pl.ds(0, CHUNK_A)]],
                              buf.at[pl.ds(0, CHUNK_A)], sem).wait()
        pltpu.make_async_copy(table_hbm.at[idx_v.at[pl.ds(0, CHUNK_B)]],
                              buf.at[pl.ds(CHUNK_A, CHUNK_B)], sem).wait()

    def accum_store(b, buf):
        def rbody(r4, acc):
            r = r4 * 4
            out = []
            for k in range(NVREG):
                s = buf[r, pl.ds(LANES * k, LANES)] + buf[r + 1, pl.ds(LANES * k, LANES)]
                t = buf[r + 2, pl.ds(LANES * k, LANES)] + buf[r + 3, pl.ds(LANES * k, LANES)]
                out.append(acc[k] + (s + t))
            return tuple(out)

        zero = jnp.zeros((LANES,), jnp.float32)
        acc = lax.fori_loop(0, SEQ // 4, rbody, (zero,) * NVREG)
        for k in range(NVREG):
            out_v[b, pl.ds(LANES * k, LANES)] = acc[k] * INV_SEQ

    start(0, buf0, sem0)

    def body(t, _):
        b0 = 2 * t
        start(b0 + 1, buf1, sem1)
        wait(buf0, sem0)
        accum_store(b0, buf0)

        @pl.when(b0 + 2 < B_PER_W)
        def _():
            start(b0 + 2, buf0, sem0)

        wait(buf1, sem1)
        accum_store(b0 + 1, buf1)
        return 0

    lax.fori_loop(0, B_PER_W // 2, body, 0)

    pltpu.sync_copy(out_v, out_hbm.at[pl.ds(obase, B_PER_W)])


@jax.jit
def _encode(x1, table_lin):
    mesh = plsc.VectorSubcoreMesh(core_axis_name="c", subcore_axis_name="s")
    return pl.kernel(
        _sc_body,
        out_type=jax.ShapeDtypeStruct((BATCH, EMBED), jnp.float32),
        mesh=mesh,
        compiler_params=pltpu.CompilerParams(use_tc_tiling_on_sc=False),
        scratch_types=[
            pltpu.VMEM((IDX_PER_W,), jnp.int32),
            pltpu.VMEM((SEQ, EMBED), jnp.float32),
            pltpu.VMEM((SEQ, EMBED), jnp.float32),
            pltpu.VMEM((B_PER_W, EMBED), jnp.float32),
            pltpu.SemaphoreType.DMA,
            pltpu.SemaphoreType.DMA,
        ],
    )(x1, table_lin)


@jax.jit
def _relayout(table):
    t_t = jnp.swapaxes(table, 0, 1)
    packed = pl.pallas_call(
        _pack_body,
        grid=(NBLK,),
        in_specs=[pl.BlockSpec((EMBED, PACK_C), lambda g: (0, g))],
        out_specs=pl.BlockSpec((PACK_H, 2 * EMBED), lambda g: (g, 0)),
        out_shape=jax.ShapeDtypeStruct((NBLK * PACK_H, 2 * EMBED), jnp.float32),
    )(t_t)
    return packed.reshape(VOCAB_PAD, EMBED)


def kernel(x, table):
    x1 = x.astype(jnp.int32).reshape(BATCH * SEQ)
    return _encode(x1, _relayout(table))


# speedup vs baseline: 3.3348x; 1.1685x over previous
import jax
import jax.numpy as jnp
from jax import lax
from jax.experimental import pallas as pl
from jax.experimental.pallas import tpu as pltpu
from jax.experimental.pallas import tpu_sc as plsc

BATCH = 4096
SEQ = 200
EMBED = 64
VOCAB = 1000000
LANES = 16

NUM_CORES = 2
NUM_SUBCORES = 16
NW = NUM_CORES * NUM_SUBCORES
B_PER_W = BATCH // NW
CHUNK_A = 104
CHUNK_B = SEQ - CHUNK_A
IDX_PER_W = B_PER_W * SEQ
NVREG = EMBED // LANES
INV_SEQ = 1.0 / SEQ

PACK_C = 8192
PACK_H = PACK_C // 2
NBLK = -(-VOCAB // PACK_C)
VOCAB_PAD = NBLK * PACK_C


def _pack_body(t_ref, out_ref):
    ta = jnp.transpose(t_ref[:, :PACK_H], (1, 0))
    tb = jnp.transpose(t_ref[:, PACK_H:], (1, 0))
    out_ref[...] = jnp.concatenate([ta, tb], axis=1)


def _sc_body(x_hbm, table_hbm, out_hbm, idx_v, buf0, buf1, out_v, sem0, sem1):
    wid = lax.axis_index("s") * NUM_CORES + lax.axis_index("c")
    obase = wid * B_PER_W

    pltpu.sync_copy(x_hbm.at[pl.ds(wid * IDX_PER_W, IDX_PER_W)], idx_v)

    def remap(r, _):
        v = idx_v[pl.ds(r * LANES, LANES)]
        m = ((v >> 13) << 13) + ((v & (PACK_H - 1)) << 1) + ((v >> 12) & 1)
        idx_v[pl.ds(r * LANES, LANES)] = m
        return 0

    lax.fori_loop(0, IDX_PER_W // LANES, remap, 0)

    def start(b, buf, sem):
        pltpu.async_copy(table_hbm.at[idx_v.at[pl.ds(SEQ * b, CHUNK_A)]],
                         buf.at[pl.ds(0, CHUNK_A)], sem)
        pltpu.async_copy(table_hbm.at[idx_v.at[pl.ds(SEQ * b + CHUNK_A, CHUNK_B)]],
                         buf.at[pl.ds(CHUNK_A, CHUNK_B)], sem)

    def wait(buf, sem):
        pltpu.make_async_copy(table_hbm.at[idx_v.at[pl.ds(0, CHUNK_A)]],
                              buf.at[pl.ds(0, CHUNK_A)], sem).wait()
        pltpu.make_async_copy(table_hbm.at[idx_v.at[pl.ds(0, CHUNK_B)]],
                              buf.at[pl.ds(CHUNK_A, CHUNK_B)], sem).wait()

    def accum_store(b, buf):
        def rbody(r4, acc):
            r = r4 * 4
            out = []
            for k in range(NVREG):
                s = buf[r, pl.ds(LANES * k, LANES)] + buf[r + 1, pl.ds(LANES * k, LANES)]
                t = buf[r + 2, pl.ds(LANES * k, LANES)] + buf[r + 3, pl.ds(LANES * k, LANES)]
                out.append(acc[k] + (s + t))
            return tuple(out)

        zero = jnp.zeros((LANES,), jnp.float32)
        acc = lax.fori_loop(0, SEQ // 4, rbody, (zero,) * NVREG)
        for k in range(NVREG):
            out_v[b, pl.ds(LANES * k, LANES)] = acc[k] * INV_SEQ

    start(0, buf0, sem0)

    def body(t, _):
        b0 = 2 * t
        start(b0 + 1, buf1, sem1)
        wait(buf0, sem0)
        accum_store(b0, buf0)

        @pl.when(b0 + 2 < B_PER_W)
        def _():
            start(b0 + 2, buf0, sem0)

        wait(buf1, sem1)
        accum_store(b0 + 1, buf1)
        return 0

    lax.fori_loop(0, B_PER_W // 2, body, 0)

    pltpu.sync_copy(out_v, out_hbm.at[pl.ds(obase, B_PER_W)])


@jax.jit
def _encode(x1, table_lin):
    mesh = plsc.VectorSubcoreMesh(core_axis_name="c", subcore_axis_name="s")
    return pl.kernel(
        _sc_body,
        out_type=jax.ShapeDtypeStruct((BATCH, EMBED), jnp.float32),
        mesh=mesh,
        compiler_params=pltpu.CompilerParams(use_tc_tiling_on_sc=False),
        scratch_types=[
            pltpu.VMEM((IDX_PER_W,), jnp.int32),
            pltpu.VMEM((SEQ, EMBED), jnp.float32),
            pltpu.VMEM((SEQ, EMBED), jnp.float32),
            pltpu.VMEM((B_PER_W, EMBED), jnp.float32),
            pltpu.SemaphoreType.DMA,
            pltpu.SemaphoreType.DMA,
        ],
    )(x1, table_lin)


@jax.jit
def _relayout(table):
    t_t = jnp.swapaxes(table, 0, 1)
    packed = pl.pallas_call(
        _pack_body,
        grid=(NBLK,),
        in_specs=[pl.BlockSpec((EMBED, PACK_C), lambda g: (0, g))],
        out_specs=pl.BlockSpec((PACK_H, 2 * EMBED), lambda g: (g, 0)),
        out_shape=jax.ShapeDtypeStruct((NBLK * PACK_H, 2 * EMBED), jnp.float32),
    )(t_t)
    return packed.reshape(VOCAB_PAD, EMBED)


def kernel(x, table):
    x1 = x.astype(jnp.int32).reshape(BATCH * SEQ)
    return _encode(x1, _relayout(table))


# speedup vs baseline: 3.6230x; 1.0864x over previous
import jax
import jax.numpy as jnp
from jax import lax
from jax.experimental import pallas as pl
from jax.experimental.pallas import tpu as pltpu
from jax.experimental.pallas import tpu_sc as plsc

BATCH = 4096
SEQ = 200
EMBED = 64
VOCAB = 1000000
LANES = 16

NUM_CORES = 2
NUM_SUBCORES = 16
NW = NUM_CORES * NUM_SUBCORES
B_PER_W = BATCH // NW
CHUNK_A = 104
CHUNK_B = SEQ - CHUNK_A
IDX_PER_W = B_PER_W * SEQ
NVREG = EMBED // LANES
INV_SEQ = 1.0 / SEQ

PACK_C = 16384
PACK_H = PACK_C // 2
NBLK = -(-VOCAB // PACK_C)
VOCAB_PAD = NBLK * PACK_C


def _pack_body(t_ref, out_ref):
    ta = jnp.transpose(t_ref[:, :PACK_H], (1, 0))
    tb = jnp.transpose(t_ref[:, PACK_H:], (1, 0))
    out_ref[...] = jnp.concatenate([ta, tb], axis=1)


def _sc_body(x_hbm, table_hbm, out_hbm, idx_v, buf0, buf1, out_v, sem0, sem1):
    wid = lax.axis_index("s") * NUM_CORES + lax.axis_index("c")
    obase = wid * B_PER_W

    pltpu.sync_copy(x_hbm.at[pl.ds(wid * IDX_PER_W, IDX_PER_W)], idx_v)

    def remap(r, _):
        v = idx_v[pl.ds(r * LANES, LANES)]
        m = ((v >> 14) << 14) + ((v & (PACK_H - 1)) << 1) + ((v >> 13) & 1)
        idx_v[pl.ds(r * LANES, LANES)] = m
        return 0

    lax.fori_loop(0, IDX_PER_W // LANES, remap, 0)

    def start(b, buf, sem):
        pltpu.async_copy(table_hbm.at[idx_v.at[pl.ds(SEQ * b, CHUNK_A)]],
                         buf.at[pl.ds(0, CHUNK_A)], sem)
        pltpu.async_copy(table_hbm.at[idx_v.at[pl.ds(SEQ * b + CHUNK_A, CHUNK_B)]],
                         buf.at[pl.ds(CHUNK_A, CHUNK_B)], sem)

    def wait(buf, sem):
        pltpu.make_async_copy(table_hbm.at[idx_v.at[pl.ds(0, CHUNK_A)]],
                              buf.at[pl.ds(0, CHUNK_A)], sem).wait()
        pltpu.make_async_copy(table_hbm.at[idx_v.at[pl.ds(0, CHUNK_B)]],
                              buf.at[pl.ds(CHUNK_A, CHUNK_B)], sem).wait()

    def accum_store(b, buf):
        def rbody(r4, acc):
            r = r4 * 4
            out = []
            for k in range(NVREG):
                s = buf[r, pl.ds(LANES * k, LANES)] + buf[r + 1, pl.ds(LANES * k, LANES)]
                t = buf[r + 2, pl.ds(LANES * k, LANES)] + buf[r + 3, pl.ds(LANES * k, LANES)]
                out.append(acc[k] + (s + t))
            return tuple(out)

        zero = jnp.zeros((LANES,), jnp.float32)
        acc = lax.fori_loop(0, SEQ // 4, rbody, (zero,) * NVREG)
        for k in range(NVREG):
            out_v[b, pl.ds(LANES * k, LANES)] = acc[k] * INV_SEQ

    start(0, buf0, sem0)

    def body(t, _):
        b0 = 2 * t
        start(b0 + 1, buf1, sem1)
        wait(buf0, sem0)
        accum_store(b0, buf0)

        @pl.when(b0 + 2 < B_PER_W)
        def _():
            start(b0 + 2, buf0, sem0)

        wait(buf1, sem1)
        accum_store(b0 + 1, buf1)
        return 0

    lax.fori_loop(0, B_PER_W // 2, body, 0)

    pltpu.sync_copy(out_v, out_hbm.at[pl.ds(obase, B_PER_W)])


@jax.jit
def _encode(x1, table_lin):
    mesh = plsc.VectorSubcoreMesh(core_axis_name="c", subcore_axis_name="s")
    return pl.kernel(
        _sc_body,
        out_type=jax.ShapeDtypeStruct((BATCH, EMBED), jnp.float32),
        mesh=mesh,
        compiler_params=pltpu.CompilerParams(use_tc_tiling_on_sc=False),
        scratch_types=[
            pltpu.VMEM((IDX_PER_W,), jnp.int32),
            pltpu.VMEM((SEQ, EMBED), jnp.float32),
            pltpu.VMEM((SEQ, EMBED), jnp.float32),
            pltpu.VMEM((B_PER_W, EMBED), jnp.float32),
            pltpu.SemaphoreType.DMA,
            pltpu.SemaphoreType.DMA,
        ],
    )(x1, table_lin)


@jax.jit
def _relayout(table):
    t_t = jnp.swapaxes(table, 0, 1)
    packed = pl.pallas_call(
        _pack_body,
        grid=(NBLK,),
        in_specs=[pl.BlockSpec((EMBED, PACK_C), lambda g: (0, g))],
        out_specs=pl.BlockSpec((PACK_H, 2 * EMBED), lambda g: (g, 0)),
        out_shape=jax.ShapeDtypeStruct((NBLK * PACK_H, 2 * EMBED), jnp.float32),
    )(t_t)
    return packed.reshape(VOCAB_PAD, EMBED)


def kernel(x, table):
    x1 = x.astype(jnp.int32).reshape(BATCH * SEQ)
    return _encode(x1, _relayout(table))


# speedup vs baseline: 3.7785x; 1.0429x over previous
import jax
import jax.numpy as jnp
from jax import lax
from jax.experimental import pallas as pl
from jax.experimental.pallas import tpu as pltpu
from jax.experimental.pallas import tpu_sc as plsc

BATCH = 4096
SEQ = 200
EMBED = 64
VOCAB = 1000000
LANES = 16

NUM_CORES = 2
NUM_SUBCORES = 16
NW = NUM_CORES * NUM_SUBCORES
B_PER_W = BATCH // NW
CHUNK_A = 104
CHUNK_B = SEQ - CHUNK_A
IDX_PER_W = B_PER_W * SEQ
NVREG = EMBED // LANES
INV_SEQ = 1.0 / SEQ

PACK_C = 32768
PACK_H = PACK_C // 2
NBLK = -(-VOCAB // PACK_C)
VOCAB_PAD = NBLK * PACK_C


def _pack_body(t_ref, out_ref):
    ta = jnp.transpose(t_ref[:, :PACK_H], (1, 0))
    tb = jnp.transpose(t_ref[:, PACK_H:], (1, 0))
    out_ref[...] = jnp.concatenate([ta, tb], axis=1)


def _sc_body(x_hbm, table_hbm, out_hbm, idx_v, buf0, buf1, out_v, sem0, sem1):
    wid = lax.axis_index("s") * NUM_CORES + lax.axis_index("c")
    obase = wid * B_PER_W

    pltpu.sync_copy(x_hbm.at[pl.ds(wid * IDX_PER_W, IDX_PER_W)], idx_v)

    def remap(r, _):
        v = idx_v[pl.ds(r * LANES, LANES)]
        m = ((v >> 15) << 15) + ((v & (PACK_H - 1)) << 1) + ((v >> 14) & 1)
        idx_v[pl.ds(r * LANES, LANES)] = m
        return 0

    lax.fori_loop(0, IDX_PER_W // LANES, remap, 0)

    def start(b, buf, sem):
        pltpu.async_copy(table_hbm.at[idx_v.at[pl.ds(SEQ * b, CHUNK_A)]],
                         buf.at[pl.ds(0, CHUNK_A)], sem)
        pltpu.async_copy(table_hbm.at[idx_v.at[pl.ds(SEQ * b + CHUNK_A, CHUNK_B)]],
                         buf.at[pl.ds(CHUNK_A, CHUNK_B)], sem)

    def wait(buf, sem):
        pltpu.make_async_copy(table_hbm.at[idx_v.at[pl.ds(0, CHUNK_A)]],
                              buf.at[pl.ds(0, CHUNK_A)], sem).wait()
        pltpu.make_async_copy(table_hbm.at[idx_v.at[pl.ds(0, CHUNK_B)]],
                              buf.at[pl.ds(CHUNK_A, CHUNK_B)], sem).wait()

    def accum_store(b, buf):
        def rbody(r4, acc):
            r = r4 * 4
            out = []
            for k in range(NVREG):
                s = buf[r, pl.ds(LANES * k, LANES)] + buf[r + 1, pl.ds(LANES * k, LANES)]
                t = buf[r + 2, pl.ds(LANES * k, LANES)] + buf[r + 3, pl.ds(LANES * k, LANES)]
                out.append(acc[k] + (s + t))
            return tuple(out)

        zero = jnp.zeros((LANES,), jnp.float32)
        acc = lax.fori_loop(0, SEQ // 4, rbody, (zero,) * NVREG)
        for k in range(NVREG):
            out_v[b, pl.ds(LANES * k, LANES)] = acc[k] * INV_SEQ

    start(0, buf0, sem0)

    def body(t, _):
        b0 = 2 * t
        start(b0 + 1, buf1, sem1)
        wait(buf0, sem0)
        accum_store(b0, buf0)

        @pl.when(b0 + 2 < B_PER_W)
        def _():
            start(b0 + 2, buf0, sem0)

        wait(buf1, sem1)
        accum_store(b0 + 1, buf1)
        return 0

    lax.fori_loop(0, B_PER_W // 2, body, 0)

    pltpu.sync_copy(out_v, out_hbm.at[pl.ds(obase, B_PER_W)])


@jax.jit
def _encode(x1, table_lin):
    mesh = plsc.VectorSubcoreMesh(core_axis_name="c", subcore_axis_name="s")
    return pl.kernel(
        _sc_body,
        out_type=jax.ShapeDtypeStruct((BATCH, EMBED), jnp.float32),
        mesh=mesh,
        compiler_params=pltpu.CompilerParams(use_tc_tiling_on_sc=False),
        scratch_types=[
            pltpu.VMEM((IDX_PER_W,), jnp.int32),
            pltpu.VMEM((SEQ, EMBED), jnp.float32),
            pltpu.VMEM((SEQ, EMBED), jnp.float32),
            pltpu.VMEM((B_PER_W, EMBED), jnp.float32),
            pltpu.SemaphoreType.DMA,
            pltpu.SemaphoreType.DMA,
        ],
    )(x1, table_lin)


@jax.jit
def _relayout(table):
    t_t = jnp.swapaxes(table, 0, 1)
    packed = pl.pallas_call(
        _pack_body,
        grid=(NBLK,),
        in_specs=[pl.BlockSpec((EMBED, PACK_C), lambda g: (0, g))],
        out_specs=pl.BlockSpec((PACK_H, 2 * EMBED), lambda g: (g, 0)),
        out_shape=jax.ShapeDtypeStruct((NBLK * PACK_H, 2 * EMBED), jnp.float32),
    )(t_t)
    return packed.reshape(VOCAB_PAD, EMBED)


def kernel(x, table):
    x1 = x.astype(jnp.int32).reshape(BATCH * SEQ)
    return _encode(x1, _relayout(table))
